# baseline (speedup 1.0000x reference)
import functools

import jax
import jax.numpy as jnp
from jax import lax
from jax.experimental import pallas as pl
from jax.experimental.pallas import tpu as pltpu
from jax.experimental.pallas import tpu_sc as plsc

B, L = 1024, 200
N = B * L
IN_W = 41
OUT_W = 160
DIM = 128
NDIM = 16

_INFO = plsc.get_sparse_core_info()
NC, NS, LANES = _INFO.num_cores, _INFO.num_subcores, _INFO.num_lanes
NW = NC * NS
TPW = N // NW
C = 80
NCHUNK = TPW // C
GPC = C // LANES


def _body(in_hbm, atom_hbm, num_hbm, out_hbm,
          atom_v, num_v, in_v0, in_v1, out_v0, out_v1,
          si0, si1, so0, so1):
    wid = lax.axis_index("s") * NC + lax.axis_index("c")
    tw = wid * TPW

    pltpu.sync_copy(atom_hbm, atom_v)
    pltpu.sync_copy(num_hbm, num_v)

    iota = lax.iota(jnp.int32, LANES)

    def in_copy(ci, iv, sem):
        return pltpu.make_async_copy(
            in_hbm.at[pl.ds(tw + ci * C, C), :], iv, sem)

    def out_copy(ci, ov, sem):
        return pltpu.make_async_copy(
            ov, out_hbm.at[pl.ds(tw + ci * C, C), :], sem)

    def compute(iv, ov):
        def group_body(g, _):
            tv = iota + g * LANES
            zeros = iota * 0

            names = plsc.load_gather(iv, [tv, zeros]).astype(jnp.int32)
            abase = names * DIM
            nbases = []
            for j in range(8):
                nid = plsc.load_gather(iv, [tv, zeros + (33 + j)])
                nbases.append(nid.astype(jnp.int32) * NDIM)

            for c in range(NDIM):
                mv = (tv + c) & (LANES - 1)
                amv = abase + mv
                avs = [plsc.load_gather(atom_v, [amv + blk * NDIM])
                       for blk in range(8)]
                nvs = [plsc.load_gather(num_v, [nbases[blk] + mv])
                       for blk in range(8)]
                p0 = plsc.load_gather(iv, [tv, mv + 4])
                sv = jnp.where(mv <= 12, mv + 20, mv - 12)
                p1 = plsc.load_gather(iv, [tv, sv])
                for blk in range(8):
                    plsc.store_scatter(ov, [tv, mv + blk * NDIM],
                                       avs[blk] + nvs[blk])
                plsc.store_scatter(ov, [tv, mv + DIM], p0)
                plsc.store_scatter(ov, [tv, mv + DIM + NDIM], p1)
            return _

        lax.fori_loop(0, GPC, group_body, None)

    bufs = ((in_v0, out_v0, si0, so0), (in_v1, out_v1, si1, so1))

    in_copy(0, in_v0, si0).start()
    in_copy(1, in_v1, si1).start()

    def super_body(i, _):
        for b, (iv, ov, sin, son) in enumerate(bufs):
            ci = 2 * i + b
            in_copy(ci, iv, sin).wait()

            @pl.when(i > 0)
            def _drain():
                out_copy(ci - 2, ov, son).wait()

            compute(iv, ov)
            out_copy(ci, ov, son).start()

            @pl.when(ci + 2 < NCHUNK)
            def _prefetch():
                in_copy(ci + 2, iv, sin).start()
        return _

    lax.fori_loop(0, NCHUNK // 2, super_body, None)

    out_copy(NCHUNK - 2, out_v0, so0).wait()
    out_copy(NCHUNK - 1, out_v1, so1).wait()


def kernel(inputs, atom_table, num_table):
    mesh = plsc.VectorSubcoreMesh(core_axis_name="c", subcore_axis_name="s")
    run = functools.partial(
        pl.kernel,
        mesh=mesh,
        compiler_params=pltpu.CompilerParams(needs_layout_passes=False),
        out_type=jax.ShapeDtypeStruct((N, OUT_W), jnp.float32),
        scratch_types=[
            pltpu.VMEM((100 * DIM,), jnp.float32),
            pltpu.VMEM((500 * NDIM,), jnp.float32),
            pltpu.VMEM((C, IN_W), jnp.float32),
            pltpu.VMEM((C, IN_W), jnp.float32),
            pltpu.VMEM((C, OUT_W), jnp.float32),
            pltpu.VMEM((C, OUT_W), jnp.float32),
            pltpu.SemaphoreType.DMA,
            pltpu.SemaphoreType.DMA,
            pltpu.SemaphoreType.DMA,
            pltpu.SemaphoreType.DMA,
        ],
    )(_body)
    out = run(inputs.reshape(N, IN_W),
              atom_table.reshape(-1),
              num_table.reshape(-1))
    return out.reshape(B, L, OUT_W)


# speedup vs baseline: 1.3606x; 1.3606x over previous
import functools

import jax
import jax.numpy as jnp
from jax import lax
from jax.experimental import pallas as pl
from jax.experimental.pallas import tpu as pltpu
from jax.experimental.pallas import tpu_sc as plsc

B, L = 1024, 200
N = B * L
IN_W = 41
OUT_W = 160
DIM = 128
NDIM = 16

_INFO = plsc.get_sparse_core_info()
NC, NS, LANES = _INFO.num_cores, _INFO.num_subcores, _INFO.num_lanes
NW = NC * NS
TPW = N // NW
C = 80
NCHUNK = TPW // C
GPC = C // LANES
K = 8


def _body(in_hbm, atom_hbm, num_hbm, out_hbm,
          atom_v, num_v, in_v0, in_v1, out_v0, out_v1, mtab,
          si0, si1, so0, so1):
    wid = lax.axis_index("s") * NC + lax.axis_index("c")
    tw = wid * TPW

    pltpu.sync_copy(atom_hbm, atom_v)
    pltpu.sync_copy(num_hbm, num_v)

    iota = lax.iota(jnp.int32, LANES)
    for c in range(LANES):
        mtab[pl.ds(c * LANES, LANES)] = (iota + c) & (LANES - 1)

    def in_copy(ci, iv, sem):
        return pltpu.make_async_copy(
            in_hbm.at[pl.ds(tw + ci * C, C), :], iv, sem)

    def out_copy(ci, ov, sem):
        return pltpu.make_async_copy(
            ov, out_hbm.at[pl.ds(tw + ci * C, C), :], sem)

    def compute(iv, ov):
        def group_body(g, _):
            tv = iota + g * LANES
            zeros = tv & 0
            mvecs = [(tv + c) & (LANES - 1) for c in range(LANES)]

            names = plsc.load_gather(iv, [tv, zeros]).astype(jnp.int32)
            abase = names * DIM
            nbases = []
            for j in range(8):
                nid = plsc.load_gather(iv, [tv, zeros + (33 + j)])
                nbases.append(nid.astype(jnp.int32) * NDIM)

            for c0 in (0, 8):
                mvs = [mtab[pl.ds((c0 + k) * LANES, LANES)] for k in range(8)]
                for blk in range(DIM // NDIM):
                    nb = nbases[blk]
                    ab = abase + blk * NDIM
                    avs = [plsc.load_gather(atom_v, [ab + mvs[k]])
                           for k in range(8)]
                    nvs = [plsc.load_gather(num_v, [nb + mvs[k]])
                           for k in range(8)]
                    for k in range(8):
                        plsc.store_scatter(ov, [tv, mvs[k] + blk * NDIM],
                                           avs[k] + nvs[k])
                p0s = [plsc.load_gather(iv, [tv, mvs[k] + 4])
                       for k in range(8)]
                svs = [jnp.where(mvs[k] <= 12, mvs[k] + 20, mvs[k] - 12)
                       for k in range(8)]
                p1s = [plsc.load_gather(iv, [tv, svs[k]]) for k in range(8)]
                for k in range(8):
                    plsc.store_scatter(ov, [tv, mvs[k] + DIM], p0s[k])
                    plsc.store_scatter(ov, [tv, mvs[k] + DIM + NDIM], p1s[k])
            return _

        lax.fori_loop(0, GPC, group_body, None)

    bufs = ((in_v0, out_v0, si0, so0), (in_v1, out_v1, si1, so1))

    in_copy(0, in_v0, si0).start()
    in_copy(1, in_v1, si1).start()

    def super_body(i, _):
        for b, (iv, ov, sin, son) in enumerate(bufs):
            ci = 2 * i + b
            in_copy(ci, iv, sin).wait()

            @pl.when(i > 0)
            def _drain():
                out_copy(ci - 2, ov, son).wait()

            compute(iv, ov)
            out_copy(ci, ov, son).start()

            @pl.when(ci + 2 < NCHUNK)
            def _prefetch():
                in_copy(ci + 2, iv, sin).start()
        return _

    lax.fori_loop(0, NCHUNK // 2, super_body, None)

    out_copy(NCHUNK - 2, out_v0, so0).wait()
    out_copy(NCHUNK - 1, out_v1, so1).wait()


def kernel(inputs, atom_table, num_table):
    mesh = plsc.VectorSubcoreMesh(core_axis_name="c", subcore_axis_name="s")
    run = functools.partial(
        pl.kernel,
        mesh=mesh,
        compiler_params=pltpu.CompilerParams(needs_layout_passes=False),
        out_type=jax.ShapeDtypeStruct((N, OUT_W), jnp.float32),
        scratch_types=[
            pltpu.VMEM((100 * DIM,), jnp.float32),
            pltpu.VMEM((500 * NDIM,), jnp.float32),
            pltpu.VMEM((C, IN_W), jnp.float32),
            pltpu.VMEM((C, IN_W), jnp.float32),
            pltpu.VMEM((C, OUT_W), jnp.float32),
            pltpu.VMEM((C, OUT_W), jnp.float32),
            pltpu.VMEM((16 * LANES,), jnp.int32),
            pltpu.SemaphoreType.DMA,
            pltpu.SemaphoreType.DMA,
            pltpu.SemaphoreType.DMA,
            pltpu.SemaphoreType.DMA,
        ],
    )(_body)
    out = run(inputs.reshape(N, IN_W),
              atom_table.reshape(-1),
              num_table.reshape(-1))
    return out.reshape(B, L, OUT_W)
